# baseline (speedup 1.0000x reference)
import functools
import math

import jax
import jax.numpy as jnp
from jax import lax
from jax.experimental import pallas as pl
from jax.experimental.pallas import tpu as pltpu
from jax.experimental.pallas import tpu_sc as plsc

D_MODEL = 128
SCALE = math.sqrt(float(D_MODEL))
NUM_WORKERS = 32
CHUNK = 128
LANES = 16
NBUF = 2


def _make_kernel(n_rows: int):
    rows_per_worker = n_rows // NUM_WORKERS
    n_chunks = rows_per_worker // CHUNK
    assert rows_per_worker % CHUNK == 0 and n_chunks >= 2 * NBUF + 2
    n_step = n_chunks - 2 * NBUF
    n_steady = n_step // NBUF
    n_rem = n_step % NBUF
    mesh = plsc.VectorSubcoreMesh(core_axis_name="c", subcore_axis_name="s")

    @functools.partial(
        pl.kernel,
        out_type=jax.ShapeDtypeStruct((n_rows, D_MODEL), jnp.float32),
        mesh=mesh,
        scratch_types=[
            pltpu.VMEM((rows_per_worker,), jnp.int32),
            [pltpu.VMEM((CHUNK, D_MODEL), jnp.float32) for _ in range(NBUF)],
            [pltpu.VMEM((CHUNK, D_MODEL), jnp.float32) for _ in range(NBUF)],
            [pltpu.SemaphoreType.DMA for _ in range(NBUF)],
            [pltpu.SemaphoreType.DMA for _ in range(NBUF)],
        ],
    )
    def gather_scale(x_hbm, table_hbm, out_hbm, idx_v, bin, bout, gsem, ssem):
        wid = lax.axis_index("s") * 2 + lax.axis_index("c")
        base = wid * rows_per_worker
        pltpu.sync_copy(x_hbm.at[pl.ds(base, rows_per_worker)], idx_v)

        def sg(ci, b):
            pltpu.async_copy(table_hbm.at[idx_v.at[pl.ds(ci * CHUNK, CHUNK)]],
                             bin[b], gsem[b])

        def wg(ci, b):
            pltpu.make_async_copy(table_hbm.at[idx_v.at[pl.ds(ci * CHUNK, CHUNK)]],
                                  bin[b], gsem[b]).wait()

        def ss(ci, b):
            pltpu.async_copy(bout[b], out_hbm.at[pl.ds(base + ci * CHUNK, CHUNK)],
                             ssem[b])

        def ws(b):
            pltpu.make_async_copy(bout[b], out_hbm.at[pl.ds(base, CHUNK)],
                                  ssem[b]).wait()

        def scale(b):
            def row(i, _):
                for j in range(D_MODEL // LANES):
                    sl = pl.ds(j * LANES, LANES)
                    bout[b][i, sl] = bin[b][i, sl] * SCALE
                return 0

            lax.fori_loop(0, CHUNK, row, 0)

        def step(ci, b, first=False):
            wg(ci, b)
            if not first:
                ws(b)
            scale(b)
            sg(ci + NBUF, b)
            ss(ci, b)

        for b in range(NBUF):
            sg(b, b)
        for b in range(NBUF):
            step(b, b, first=True)

        def group(g, _):
            ci0 = g * NBUF
            for k in range(NBUF):
                step(ci0 + k, k)
            return 0

        lax.fori_loop(1, 1 + n_steady, group, 0)

        ci0 = (1 + n_steady) * NBUF
        for k in range(n_rem):
            step(ci0 + k, (ci0 + k) % NBUF)
        for k in range(n_rem, n_rem + NBUF):
            ci = ci0 + k
            b = ci % NBUF
            wg(ci, b)
            ws(b)
            scale(b)
            ss(ci, b)
        for ci in range(n_chunks - NBUF, n_chunks):
            ws(ci % NBUF)

    return gather_scale


def kernel(x, table):
    b, s = x.shape
    n_rows = b * s
    out = _make_kernel(n_rows)(x.reshape(n_rows).astype(jnp.int32), table)
    return out.reshape(b, s, D_MODEL)


# speedup vs baseline: 1.0063x; 1.0063x over previous
import functools
import math

import jax
import jax.numpy as jnp
from jax import lax
from jax.experimental import pallas as pl
from jax.experimental.pallas import tpu as pltpu
from jax.experimental.pallas import tpu_sc as plsc

D_MODEL = 128
SCALE = math.sqrt(float(D_MODEL))
NUM_WORKERS = 32
CHUNK = 128
LANES = 16
NBUF = 3


def _make_kernel(n_rows: int):
    rows_per_worker = n_rows // NUM_WORKERS
    n_chunks = rows_per_worker // CHUNK
    assert rows_per_worker % CHUNK == 0 and n_chunks >= 2 * NBUF + 2
    n_step = n_chunks - 2 * NBUF
    n_steady = n_step // NBUF
    n_rem = n_step % NBUF
    mesh = plsc.VectorSubcoreMesh(core_axis_name="c", subcore_axis_name="s")

    @functools.partial(
        pl.kernel,
        out_type=jax.ShapeDtypeStruct((n_rows, D_MODEL), jnp.float32),
        mesh=mesh,
        scratch_types=[
            pltpu.VMEM((rows_per_worker,), jnp.int32),
            [pltpu.VMEM((CHUNK, D_MODEL), jnp.float32) for _ in range(NBUF)],
            [pltpu.VMEM((CHUNK, D_MODEL), jnp.float32) for _ in range(NBUF)],
            [pltpu.SemaphoreType.DMA for _ in range(NBUF)],
            [pltpu.SemaphoreType.DMA for _ in range(NBUF)],
        ],
    )
    def gather_scale(x_hbm, table_hbm, out_hbm, idx_v, bin, bout, gsem, ssem):
        wid = lax.axis_index("s") * 2 + lax.axis_index("c")
        base = wid * rows_per_worker
        pltpu.sync_copy(x_hbm.at[pl.ds(base, rows_per_worker)], idx_v)

        def sg(ci, b):
            pltpu.async_copy(table_hbm.at[idx_v.at[pl.ds(ci * CHUNK, CHUNK)]],
                             bin[b], gsem[b])

        def wg(ci, b):
            pltpu.make_async_copy(table_hbm.at[idx_v.at[pl.ds(ci * CHUNK, CHUNK)]],
                                  bin[b], gsem[b]).wait()

        def ss(ci, b):
            pltpu.async_copy(bout[b], out_hbm.at[pl.ds(base + ci * CHUNK, CHUNK)],
                             ssem[b])

        def ws(b):
            pltpu.make_async_copy(bout[b], out_hbm.at[pl.ds(base, CHUNK)],
                                  ssem[b]).wait()

        def scale(b):
            def rows(i, _):
                for r in range(2):
                    for j in range(D_MODEL // LANES):
                        sl = pl.ds(j * LANES, LANES)
                        bout[b][i * 2 + r, sl] = bin[b][i * 2 + r, sl] * SCALE
                return 0

            lax.fori_loop(0, CHUNK // 2, rows, 0)

        def step(ci, b, first=False):
            wg(ci, b)
            if not first:
                ws(b)
            scale(b)
            sg(ci + NBUF, b)
            ss(ci, b)

        for b in range(NBUF):
            sg(b, b)
        for b in range(NBUF):
            step(b, b, first=True)

        def group(g, _):
            ci0 = g * NBUF
            for k in range(NBUF):
                step(ci0 + k, k)
            return 0

        lax.fori_loop(1, 1 + n_steady, group, 0)

        ci0 = (1 + n_steady) * NBUF
        for k in range(n_rem):
            step(ci0 + k, (ci0 + k) % NBUF)
        for k in range(n_rem, n_rem + NBUF):
            ci = ci0 + k
            b = ci % NBUF
            wg(ci, b)
            ws(b)
            scale(b)
            ss(ci, b)
        for ci in range(n_chunks - NBUF, n_chunks):
            ws(ci % NBUF)

    return gather_scale


def kernel(x, table):
    b, s = x.shape
    n_rows = b * s
    out = _make_kernel(n_rows)(x.reshape(n_rows).astype(jnp.int32), table)
    return out.reshape(b, s, D_MODEL)


# speedup vs baseline: 1.0080x; 1.0017x over previous
import functools
import math

import jax
import jax.numpy as jnp
from jax import lax
from jax.experimental import pallas as pl
from jax.experimental.pallas import tpu as pltpu
from jax.experimental.pallas import tpu_sc as plsc

D_MODEL = 128
SCALE = math.sqrt(float(D_MODEL))
NUM_WORKERS = 32
CHUNK = 128
LANES = 16
NBUF = 3


def _make_kernel(n_rows: int):
    rows_per_worker = n_rows // NUM_WORKERS
    n_chunks = rows_per_worker // CHUNK
    assert rows_per_worker % CHUNK == 0 and n_chunks >= 2 * NBUF + 2
    n_step = n_chunks - 2 * NBUF
    n_steady = n_step // NBUF
    n_rem = n_step % NBUF
    mesh = plsc.VectorSubcoreMesh(core_axis_name="c", subcore_axis_name="s")

    @functools.partial(
        pl.kernel,
        out_type=jax.ShapeDtypeStruct((n_rows, D_MODEL), jnp.float32),
        mesh=mesh,
        scratch_types=[
            pltpu.VMEM((rows_per_worker,), jnp.int32),
            [pltpu.VMEM((CHUNK, D_MODEL), jnp.float32) for _ in range(NBUF)],
            [pltpu.VMEM((CHUNK, D_MODEL), jnp.float32) for _ in range(NBUF)],
            [pltpu.SemaphoreType.DMA for _ in range(NBUF)],
            [pltpu.SemaphoreType.DMA for _ in range(NBUF)],
        ],
    )
    def gather_scale(x_hbm, table_hbm, out_hbm, idx_v, bin, bout, gsem, ssem):
        wid = lax.axis_index("s") * 2 + lax.axis_index("c")
        base = wid * rows_per_worker
        pltpu.sync_copy(x_hbm.at[pl.ds(base, rows_per_worker)], idx_v)

        def sg(ci, b):
            pltpu.async_copy(table_hbm.at[idx_v.at[pl.ds(ci * CHUNK, CHUNK)]],
                             bin[b], gsem[b])

        def wg(ci, b):
            pltpu.make_async_copy(table_hbm.at[idx_v.at[pl.ds(ci * CHUNK, CHUNK)]],
                                  bin[b], gsem[b]).wait()

        def ss(ci, b):
            pltpu.async_copy(bout[b], out_hbm.at[pl.ds(base + ci * CHUNK, CHUNK)],
                             ssem[b])

        def ws(b):
            pltpu.make_async_copy(bout[b], out_hbm.at[pl.ds(base, CHUNK)],
                                  ssem[b]).wait()

        def scale(b):
            def row(i, _):
                for j in range(D_MODEL // LANES):
                    sl = pl.ds(j * LANES, LANES)
                    bout[b][i, sl] = bin[b][i, sl] * SCALE
                return 0

            lax.fori_loop(0, CHUNK, row, 0)

        def step(ci, b, first=False):
            wg(ci, b)
            if not first:
                ws(b)
            scale(b)
            sg(ci + NBUF, b)
            ss(ci, b)

        for b in range(NBUF):
            sg(b, b)
        for b in range(NBUF):
            step(b, b, first=True)

        def group(g, _):
            ci0 = g * NBUF
            for k in range(NBUF):
                step(ci0 + k, k)
            return 0

        lax.fori_loop(1, 1 + n_steady, group, 0)

        ci0 = (1 + n_steady) * NBUF
        for k in range(n_rem):
            step(ci0 + k, (ci0 + k) % NBUF)
        for k in range(n_rem, n_rem + NBUF):
            ci = ci0 + k
            b = ci % NBUF
            wg(ci, b)
            ws(b)
            scale(b)
            ss(ci, b)
        for ci in range(n_chunks - NBUF, n_chunks):
            ws(ci % NBUF)

    return gather_scale


def kernel(x, table):
    b, s = x.shape
    n_rows = b * s
    out = _make_kernel(n_rows)(x.reshape(n_rows).astype(jnp.int32), table)
    return out.reshape(b, s, D_MODEL)
